# baseline (speedup 1.0000x reference)
import jax
import jax.numpy as jnp
from jax.experimental import pallas as pl
from jax.experimental.pallas import tpu as pltpu

NE = 16384
DIM = 512
HID = 100
PRED = 2
BE = 2048
NB = NE // BE

_PREC = jax.lax.Precision.DEFAULT


def _outer(h_ref, r_ref, b2_ref, wo2_ref, bo2_ref, pos_ref, neg_ref):
    def inner(hs_ref, hp_ref, hn_ref, po_ref, no_ref):
        x = jnp.concatenate(
            [hs_ref[...].astype(jnp.bfloat16),
             hp_ref[...].astype(jnp.bfloat16),
             hn_ref[...].astype(jnp.bfloat16)], axis=1)
        z = jnp.dot(x, r_ref[...], preferred_element_type=jnp.float32,
                    precision=_PREC)
        e = jnp.maximum(z + b2_ref[...], 0.0).astype(jnp.bfloat16)
        pt = jax.lax.dot_general(wo2_ref[...], e, (((1,), (1,)), ((), ())),
                                 preferred_element_type=jnp.float32,
                                 precision=_PREC)
        bo = bo2_ref[...]
        po_ref[...] = pt[0:PRED, :] + bo
        no_ref[...] = pt[PRED:2 * PRED, :] + bo

    pltpu.emit_pipeline(
        inner,
        grid=(NB,),
        in_specs=[
            pl.BlockSpec((BE, DIM), lambda i: (i, 0)),
            pl.BlockSpec((BE, DIM), lambda i: (i + NB, 0)),
            pl.BlockSpec((BE, DIM), lambda i: (i + 2 * NB, 0)),
        ],
        out_specs=[
            pl.BlockSpec((PRED, BE), lambda i: (0, i)),
            pl.BlockSpec((PRED, BE), lambda i: (0, i)),
        ],
    )(h_ref, h_ref, h_ref, pos_ref, neg_ref)


@jax.jit
def _run(h, r, b2, wo2, bo2):
    out_shape = jax.ShapeDtypeStruct((PRED, NE), jnp.float32)
    vmem = pl.BlockSpec(memory_space=pltpu.MemorySpace.VMEM)
    pos, neg = pl.pallas_call(
        _outer,
        in_specs=[
            pl.BlockSpec(memory_space=pl.ANY),
            vmem, vmem, vmem, vmem,
        ],
        out_specs=[
            pl.BlockSpec(memory_space=pl.ANY),
            pl.BlockSpec(memory_space=pl.ANY),
        ],
        out_shape=[out_shape, out_shape],
        compiler_params=pltpu.CompilerParams(
            vmem_limit_bytes=100 * 1024 * 1024,
        ),
    )(h, r, b2, wo2, bo2)
    return pos.T, neg.T


def kernel(h, W_src, b_src, W_dst, b_dst, W_out, b_out, neg_samples):
    del neg_samples
    z100 = jnp.zeros((DIM, HID), W_src.dtype)
    r = jnp.block([[W_src, W_src], [W_dst, z100], [z100, W_dst]])
    b_sum = (b_src + b_dst)
    b2 = jnp.concatenate([b_sum, b_sum]).reshape(1, 2 * HID)
    z2 = jnp.zeros((HID, PRED), W_out.dtype)
    wo2 = jnp.block([[W_out, z2], [z2, W_out]]).T
    bo2 = b_out.reshape(PRED, 1)
    return _run(h, r.astype(jnp.bfloat16), b2, wo2.astype(jnp.bfloat16), bo2)


# speedup vs baseline: 1.0044x; 1.0044x over previous
import jax
import jax.numpy as jnp
from jax.experimental import pallas as pl
from jax.experimental.pallas import tpu as pltpu

NE = 16384
DIM = 512
HID = 100
PRED = 2
BE = 2048

_PREC = jax.lax.Precision.DEFAULT


def _body(hs_ref, hp_ref, hn_ref, r_ref, b2_ref, wo2_ref, bo2_ref,
          pos_ref, neg_ref):
    x = jnp.concatenate(
        [hs_ref[...].astype(jnp.bfloat16),
         hp_ref[...].astype(jnp.bfloat16),
         hn_ref[...].astype(jnp.bfloat16)], axis=1)
    z = jnp.dot(x, r_ref[...], preferred_element_type=jnp.float32,
                precision=_PREC)
    e = jnp.maximum(z + b2_ref[...], 0.0).astype(jnp.bfloat16)
    pt = jax.lax.dot_general(wo2_ref[...], e, (((1,), (1,)), ((), ())),
                             preferred_element_type=jnp.float32,
                             precision=_PREC)
    bo = bo2_ref[...]
    pos_ref[...] = pt[0:PRED, :] + bo
    neg_ref[...] = pt[PRED:2 * PRED, :] + bo


@jax.jit
def _run(h, r, b2, wo2, bo2):
    nb = NE // BE
    full = lambda i: (0, 0)
    out_shape = jax.ShapeDtypeStruct((PRED, NE), jnp.float32)
    pos, neg = pl.pallas_call(
        _body,
        grid=(nb,),
        in_specs=[
            pl.BlockSpec((BE, DIM), lambda i: (i, 0)),
            pl.BlockSpec((BE, DIM), lambda i: (i + nb, 0)),
            pl.BlockSpec((BE, DIM), lambda i: (i + 2 * nb, 0)),
            pl.BlockSpec((3 * DIM, 2 * HID), full),
            pl.BlockSpec((1, 2 * HID), full),
            pl.BlockSpec((2 * PRED, 2 * HID), full),
            pl.BlockSpec((PRED, 1), full),
        ],
        out_specs=[
            pl.BlockSpec((PRED, BE), lambda i: (0, i)),
            pl.BlockSpec((PRED, BE), lambda i: (0, i)),
        ],
        out_shape=[out_shape, out_shape],
        compiler_params=pltpu.CompilerParams(
            dimension_semantics=("parallel",),
            vmem_limit_bytes=100 * 1024 * 1024,
        ),
    )(h, h, h, r, b2, wo2, bo2)
    return pos.T, neg.T


def kernel(h, W_src, b_src, W_dst, b_dst, W_out, b_out, neg_samples):
    del neg_samples
    z100 = jnp.zeros((DIM, HID), W_src.dtype)
    r = jnp.block([[W_src, W_src], [W_dst, z100], [z100, W_dst]])
    b_sum = (b_src + b_dst)
    b2 = jnp.concatenate([b_sum, b_sum]).reshape(1, 2 * HID)
    z2 = jnp.zeros((HID, PRED), W_out.dtype)
    wo2 = jnp.block([[W_out, z2], [z2, W_out]]).T
    bo2 = b_out.reshape(PRED, 1)
    return _run(h, r.astype(jnp.bfloat16), b2, wo2.astype(jnp.bfloat16), bo2)
